# Initial kernel scaffold; baseline (speedup 1.0000x reference)
import jax
import jax.numpy as jnp
from jax.experimental import pallas as pl


def kernel(x, table):
    raise NotImplementedError("write your pallas kernel here")


# speedup vs baseline: 1.3074x; 1.3074x over previous
import functools

import jax
import jax.numpy as jnp
from jax import lax
from jax.experimental import pallas as pl
from jax.experimental.pallas import tpu as pltpu
from jax.experimental.pallas import tpu_sc as plsc

EMBED = 32
NC, NS = 2, 16
NW = NC * NS
CHUNK = 128


@functools.lru_cache(maxsize=None)
def _make_kernel(B: int):
    assert B % (NW * CHUNK) == 0
    n_chunks = B // (NW * CHUNK)
    mesh = plsc.VectorSubcoreMesh(core_axis_name="c", subcore_axis_name="s")

    @functools.partial(
        pl.kernel,
        out_type=jax.ShapeDtypeStruct((B, EMBED), jnp.float32),
        mesh=mesh,
        scratch_types=[
            pltpu.VMEM((n_chunks, CHUNK), jnp.int32),
            pltpu.VMEM((CHUNK, EMBED), jnp.float32),
            pltpu.SemaphoreType.DMA,
        ],
        compiler_params=pltpu.CompilerParams(use_tc_tiling_on_sc=False),
    )
    def body(idx_hbm, table_hbm, out_hbm, idx_v, rows_v, gsem):
        wid = lax.axis_index("s") * NC + lax.axis_index("c")
        chunk0 = wid * n_chunks
        row0 = chunk0 * CHUNK
        pltpu.sync_copy(idx_hbm.at[pl.ds(chunk0, n_chunks)], idx_v)

        @pl.loop(0, n_chunks)
        def _(j):
            pltpu.async_copy(table_hbm.at[idx_v.at[j]], rows_v, gsem).wait()
            pltpu.sync_copy(rows_v, out_hbm.at[pl.ds(row0 + j * CHUNK, CHUNK)])

    return body


def kernel(x, table):
    B, H = x.shape
    flat = x.reshape(B * H).astype(jnp.int32)
    idx2d = flat.reshape((B * H) // CHUNK, CHUNK)
    out = _make_kernel(B * H)(idx2d, table)
    return out.reshape(B, H, EMBED)


# speedup vs baseline: 1.4918x; 1.1410x over previous
import functools

import jax
import jax.numpy as jnp
from jax import lax
from jax.experimental import pallas as pl
from jax.experimental.pallas import tpu as pltpu
from jax.experimental.pallas import tpu_sc as plsc

EMBED = 32
NC, NS = 2, 16
NW = NC * NS
CHUNK = 128


G = 10
GROWS = G * CHUNK


@functools.lru_cache(maxsize=None)
def _make_kernel(B: int):
    assert B % (NW * CHUNK) == 0
    n_chunks = B // (NW * CHUNK)
    assert n_chunks % (2 * G) == 0
    n_groups = n_chunks // G
    mesh = plsc.VectorSubcoreMesh(core_axis_name="c", subcore_axis_name="s")

    @functools.partial(
        pl.kernel,
        out_type=jax.ShapeDtypeStruct((B, EMBED), jnp.float32),
        mesh=mesh,
        scratch_types=[
            pltpu.VMEM((n_chunks, CHUNK), jnp.int32),
            pltpu.VMEM((GROWS, EMBED), jnp.float32),
            pltpu.VMEM((GROWS, EMBED), jnp.float32),
            pltpu.SemaphoreType.DMA,
            pltpu.SemaphoreType.DMA,
            pltpu.SemaphoreType.DMA,
            pltpu.SemaphoreType.DMA,
        ],
        compiler_params=pltpu.CompilerParams(use_tc_tiling_on_sc=False),
    )
    def body(idx_hbm, table_hbm, out_hbm, idx_v, buf0, buf1,
             gsem0, gsem1, ssem0, ssem1):
        wid = lax.axis_index("s") * NC + lax.axis_index("c")
        chunk0 = wid * n_chunks
        row0 = chunk0 * CHUNK
        pltpu.sync_copy(idx_hbm.at[pl.ds(chunk0, n_chunks)], idx_v)

        def fire_gathers(grp, buf, sem):
            for c in range(G):
                pltpu.async_copy(table_hbm.at[idx_v.at[grp * G + c]],
                                 buf.at[pl.ds(c * CHUNK, CHUNK)], sem)

        def drain_gathers(buf, sem):
            pltpu.make_async_copy(out_hbm.at[pl.ds(0, GROWS)], buf, sem).wait()

        def fire_store(grp, buf, sem):
            return pltpu.async_copy(
                buf, out_hbm.at[pl.ds(row0 + grp * GROWS, GROWS)], sem)

        fire_gathers(0, buf0, gsem0)
        fire_gathers(1, buf1, gsem1)

        @pl.loop(0, n_groups - 2, step=2)
        def _(g):
            drain_gathers(buf0, gsem0)
            st0 = fire_store(g, buf0, ssem0)
            drain_gathers(buf1, gsem1)
            st1 = fire_store(g + 1, buf1, ssem1)
            st0.wait()
            fire_gathers(g + 2, buf0, gsem0)
            st1.wait()
            fire_gathers(g + 3, buf1, gsem1)

        drain_gathers(buf0, gsem0)
        st0 = fire_store(n_groups - 2, buf0, ssem0)
        drain_gathers(buf1, gsem1)
        st1 = fire_store(n_groups - 1, buf1, ssem1)
        st0.wait()
        st1.wait()

    return body


def kernel(x, table):
    B, H = x.shape
    flat = x.reshape(B * H).astype(jnp.int32)
    idx2d = flat.reshape((B * H) // CHUNK, CHUNK)
    out = _make_kernel(B * H)(idx2d, table)
    return out.reshape(B, H, EMBED)
